# baseline (speedup 1.0000x reference)
import functools

import jax
import jax.numpy as jnp
from jax import lax
from jax.experimental import pallas as pl
from jax.experimental.pallas import tpu as pltpu
from jax.experimental.pallas import tpu_sc as plsc

N = 16384
C = 20
NBINS = 8
NMC = 4
NSC = 10
NT = 32
CHS = N // NT
NV = CHS // 16

_AAT = 0
_MCP = _AAT + C
_MCL = _MCP + 3 * NMC
_SCP = _MCL + 3 * NMC
_SCL = _SCP + 3 * NSC
_MSK = _SCL + 3 * NSC
_CAP = _MSK + NSC
_CAL = _CAP + 3
_NROWS = _CAL + 3

_F32 = jnp.float32
_I32 = jnp.int32
_LN2 = 0.6931471805599453


def _rsqrt(x):
    i = lax.bitcast_convert_type(x, _I32)
    i = jnp.int32(0x5F3759DF) - lax.shift_right_arithmetic(i, 1)
    y = lax.bitcast_convert_type(i, _F32)
    for _ in range(3):
        y = y * (1.5 - 0.5 * x * y * y)
    return y


def _sqrt(x):
    xs = jnp.maximum(x, 1e-30)
    return x * _rsqrt(xs)


def _log(x):
    xi = lax.bitcast_convert_type(x, _I32)
    e = lax.shift_right_arithmetic(xi, 23) - 127
    mi = lax.bitwise_or(lax.bitwise_and(xi, jnp.int32(0x007FFFFF)),
                        jnp.int32(0x3F800000))
    m = lax.bitcast_convert_type(mi, _F32)
    big = m > 1.4142135
    m = jnp.where(big, m * 0.5, m)
    e = jnp.where(big, e + 1, e)
    t = (m - 1.0) / (m + 1.0)
    t2 = t * t
    p = 2.0 * t * (1.0 + t2 * (1.0 / 3.0 + t2 * (0.2 + t2 * (1.0 / 7.0))))
    return p + e.astype(_F32) * _LN2


def _tree_max(vs):
    while len(vs) > 1:
        nxt = [jnp.maximum(vs[i], vs[i + 1]) for i in range(0, len(vs) - 1, 2)]
        if len(vs) % 2:
            nxt.append(vs[-1])
        vs = nxt
    return vs[0]


def _tree_sum(vs):
    while len(vs) > 1:
        nxt = [vs[i] + vs[i + 1] for i in range(0, len(vs) - 1, 2)]
        if len(vs) % 2:
            nxt.append(vs[-1])
        vs = nxt
    return vs[0]


def _zeros8():
    return tuple(jnp.zeros((16,), _F32) for _ in range(8))


def _accum(accs, cnts, b, val):
    na, nc = [], []
    for k in range(8):
        msk = b == k
        na.append(accs[k] + jnp.where(msk, val, 0.0))
        nc.append(cnts[k] + jnp.where(msk, 1.0, 0.0))
    return tuple(na), tuple(nc)


def _halves(iota, lo_list, hi_list):
    vec = jnp.zeros((16,), _F32)
    for k in range(8):
        vec = jnp.where(iota == k, jnp.sum(lo_list[k]), vec)
        vec = jnp.where(iota == (k + 8), jnp.sum(hi_list[k]), vec)
    return vec


def _sc_body(aat_h, mcp_h, mcl_h, scp_h, scl_h, msk_h, cap_h, cal_h,
             lab_h, ia_h, im_h, isc_h, ic_h, out_h,
             aat_v, mcp_v, mcl_v, scp_v, scl_v, msk_v, cap_v, cal_v,
             lab_v, ia_v, im_v, isc_v, ic_v, part_v, sem):
    wid = lax.axis_index("s") * 2 + lax.axis_index("c")
    iota = lax.iota(_I32, 16)
    tok0 = wid * CHS

    d_a = [pltpu.async_copy(aat_h.at[:, pl.ds(tok0, CHS)], aat_v, sem),
           pltpu.async_copy(lab_h.at[pl.ds(tok0, CHS)], lab_v, sem),
           pltpu.async_copy(ia_h.at[pl.ds(tok0, CHS)], ia_v, sem)]
    d_m = [pltpu.async_copy(mcp_h.at[:, :, pl.ds(tok0, CHS)], mcp_v, sem),
           pltpu.async_copy(mcl_h.at[:, :, pl.ds(tok0, CHS)], mcl_v, sem),
           pltpu.async_copy(im_h.at[pl.ds(tok0, CHS)], im_v, sem)]
    d_s = [pltpu.async_copy(scp_h.at[:, :, pl.ds(tok0, CHS)], scp_v, sem),
           pltpu.async_copy(scl_h.at[:, :, pl.ds(tok0, CHS)], scl_v, sem),
           pltpu.async_copy(msk_h.at[:, pl.ds(tok0, CHS)], msk_v, sem),
           pltpu.async_copy(isc_h.at[pl.ds(tok0, CHS)], isc_v, sem)]
    d_c = [pltpu.async_copy(cap_h.at[:, pl.ds(tok0, CHS)], cap_v, sem),
           pltpu.async_copy(cal_h.at[:, pl.ds(tok0, CHS)], cal_v, sem),
           pltpu.async_copy(ic_h.at[pl.ds(tok0, CHS)], ic_v, sem)]
    for d in d_a:
        d.wait()

    @plsc.parallel_loop(0, NV, 1, unroll=2, carry=(_zeros8(), _zeros8()))
    def aat_loop(i, carry):
        accs, cnts = carry
        base = i * 16
        b = ia_v[pl.ds(base, 16)]
        lbl = lab_v[pl.ds(base, 16)]
        vals = [aat_v[c, pl.ds(base, 16)] for c in range(C)]
        mx = _tree_max(vals)
        ssum = _tree_sum([jnp.exp(v - mx) for v in vals])
        g = plsc.load_gather(aat_v, [lbl, iota + base])
        ce = mx + _log(ssum) - g
        pt = jnp.exp(-ce)
        loss = 0.25 * (1.0 - pt) * (1.0 - pt) * ce
        return _accum(accs, cnts, b, loss)

    aat_accs, aat_cnts = aat_loop
    for d in d_m:
        d.wait()

    @plsc.parallel_loop(0, NV, 1, unroll=2, carry=(_zeros8(), _zeros8()))
    def mc_loop(i, carry):
        accs, cnts = carry
        base = i * 16
        b = im_v[pl.ds(base, 16)]
        dists = []
        for a in range(NMC):
            dx = (mcp_v[0, a, pl.ds(base, 16)]
                  - mcl_v[0, a, pl.ds(base, 16)])
            dy = (mcp_v[1, a, pl.ds(base, 16)]
                  - mcl_v[1, a, pl.ds(base, 16)])
            dz = (mcp_v[2, a, pl.ds(base, 16)]
                  - mcl_v[2, a, pl.ds(base, 16)])
            dists.append(_sqrt(dx * dx + dy * dy + dz * dz))
        return _accum(accs, cnts, b, _tree_sum(dists))

    mc_accs, mc_cnts = mc_loop
    for d in d_s:
        d.wait()

    @plsc.parallel_loop(0, NV, 1, unroll=2, carry=(_zeros8(), _zeros8()))
    def sc_loop(i, carry):
        accs, cnts = carry
        base = i * 16
        b = isc_v[pl.ds(base, 16)]
        dists = []
        for a in range(NSC):
            dx = (scp_v[a, 0, pl.ds(base, 16)]
                  - scl_v[a, 0, pl.ds(base, 16)])
            dy = (scp_v[a, 1, pl.ds(base, 16)]
                  - scl_v[a, 1, pl.ds(base, 16)])
            dz = (scp_v[a, 2, pl.ds(base, 16)]
                  - scl_v[a, 2, pl.ds(base, 16)])
            mv = msk_v[a, pl.ds(base, 16)]
            dists.append(_sqrt(dx * dx + dy * dy + dz * dz) * mv)
        return _accum(accs, cnts, b, _tree_sum(dists))

    sc_accs, sc_cnts = sc_loop
    for d in d_c:
        d.wait()

    @plsc.parallel_loop(0, NV, 1, unroll=2, carry=(_zeros8(), _zeros8()))
    def ca_loop(i, carry):
        accs, cnts = carry
        base = i * 16
        b = ic_v[pl.ds(base, 16)]
        dx = cap_v[0, pl.ds(base, 16)] - cal_v[0, pl.ds(base, 16)]
        dy = cap_v[1, pl.ds(base, 16)] - cal_v[1, pl.ds(base, 16)]
        dz = cap_v[2, pl.ds(base, 16)] - cal_v[2, pl.ds(base, 16)]
        tot = _sqrt(dx * dx + dy * dy + dz * dz)
        return _accum(accs, cnts, b, tot)

    ca_accs, ca_cnts = ca_loop

    part_v[pl.ds(0, 16)] = _halves(iota, aat_accs, mc_accs)
    part_v[pl.ds(16, 16)] = _halves(iota, sc_accs, ca_accs)
    part_v[pl.ds(32, 16)] = _halves(iota, aat_cnts, mc_cnts)
    part_v[pl.ds(48, 16)] = _halves(iota, sc_cnts, ca_cnts)
    z = jnp.zeros((16,), _F32)
    part_v[pl.ds(64, 16)] = z
    part_v[pl.ds(80, 16)] = z
    part_v[pl.ds(96, 16)] = z
    part_v[pl.ds(112, 16)] = z
    pltpu.sync_copy(part_v, out_h.at[wid])


_mesh = plsc.VectorSubcoreMesh(core_axis_name="c", subcore_axis_name="s",
                               num_cores=2)

_sc_call = functools.partial(
    pl.kernel,
    out_type=jax.ShapeDtypeStruct((NT, 128), _F32),
    mesh=_mesh,
    compiler_params=pltpu.CompilerParams(
        needs_layout_passes=False,
        disable_bounds_checks=True,
        disable_semaphore_checks=True,
    ),
    scratch_types=[
        pltpu.VMEM((C, CHS), _F32),
        pltpu.VMEM((3, NMC, CHS), _F32),
        pltpu.VMEM((3, NMC, CHS), _F32),
        pltpu.VMEM((NSC, 3, CHS), _F32),
        pltpu.VMEM((NSC, 3, CHS), _F32),
        pltpu.VMEM((NSC, CHS), _F32),
        pltpu.VMEM((3, CHS), _F32),
        pltpu.VMEM((3, CHS), _F32),
        pltpu.VMEM((CHS,), _I32),
        pltpu.VMEM((CHS,), _I32),
        pltpu.VMEM((CHS,), _I32),
        pltpu.VMEM((CHS,), _I32),
        pltpu.VMEM((CHS,), _I32),
        pltpu.VMEM((128,), _F32),
        pltpu.SemaphoreType.DMA,
    ],
)(_sc_body)


def _tc_combine(p_ref, o_grad, o_aat, o_mc, o_sc, o_ca):
    x = p_ref[...]
    tot = jnp.sum(x, axis=0, keepdims=True)
    sums = tot[:, 0:32]
    cnts = tot[:, 32:64]
    means = sums / jnp.maximum(cnts, 1.0)
    aat = jnp.sum(means[:, 0:8]) * (1.0 / NBINS)
    mc = jnp.sum(means[:, 8:16]) * (1.0 / (NBINS * NMC))
    sc = jnp.sum(means[:, 16:24]) * (1.0 / (NBINS * NSC))
    ca = jnp.sum(means[:, 24:32]) * (1.0 / NBINS)
    grad = aat + ca + mc + 0.5 * sc
    o_grad[0, 0] = grad
    o_aat[0, 0] = aat
    o_mc[0, 0] = mc
    o_sc[0, 0] = sc
    o_ca[0, 0] = ca


_tc_call = pl.pallas_call(
    _tc_combine,
    out_shape=[jax.ShapeDtypeStruct((1, 1), _F32)] * 5,
    out_specs=[pl.BlockSpec(memory_space=pltpu.SMEM)] * 5,
)


def kernel(AAtype_pred, MCcoor_pred, SCcoor_pred, CAnoise_pred, AAtype_label,
           MCcoor_label, SCcoor_label, SCcoor_mask, CAnoise_label,
           AAtype_scatter, MCcoor_scatter, SCcoor_scatter, CAnoise_scatter):
    part = _sc_call(
        AAtype_pred.T,
        jnp.transpose(MCcoor_pred, (2, 1, 0)),
        jnp.transpose(MCcoor_label, (2, 1, 0)),
        jnp.transpose(SCcoor_pred, (1, 2, 0)),
        jnp.transpose(SCcoor_label, (1, 2, 0)),
        SCcoor_mask.T,
        CAnoise_pred.T,
        CAnoise_label.T,
        AAtype_label.astype(jnp.int32),
        AAtype_scatter.astype(jnp.int32),
        MCcoor_scatter.astype(jnp.int32),
        SCcoor_scatter.astype(jnp.int32),
        CAnoise_scatter.astype(jnp.int32),
    )
    grad, aat, mc, sc, ca = _tc_call(part)
    return (grad.reshape(()), aat.reshape(()), mc.reshape(()),
            sc.reshape(()), ca.reshape(()))


# speedup vs baseline: 1.1069x; 1.1069x over previous
import functools

import jax
import jax.numpy as jnp
from jax import lax
from jax.experimental import pallas as pl
from jax.experimental.pallas import tpu as pltpu
from jax.experimental.pallas import tpu_sc as plsc

N = 16384
C = 20
NBINS = 8
NMC = 4
NSC = 10
NT = 32
CHS = N // NT
NV = CHS // 16

_AAT = 0
_MCP = _AAT + C
_MCL = _MCP + 3 * NMC
_SCP = _MCL + 3 * NMC
_SCL = _SCP + 3 * NSC
_MSK = _SCL + 3 * NSC
_CAP = _MSK + NSC
_CAL = _CAP + 3
_NROWS = _CAL + 3

_F32 = jnp.float32
_I32 = jnp.int32
_LN2 = 0.6931471805599453


def _rsqrt(x):
    i = lax.bitcast_convert_type(x, _I32)
    i = jnp.int32(0x5F3759DF) - lax.shift_right_arithmetic(i, 1)
    y = lax.bitcast_convert_type(i, _F32)
    for _ in range(3):
        y = y * (1.5 - 0.5 * x * y * y)
    return y


def _sqrt(x):
    xs = jnp.maximum(x, 1e-30)
    return x * _rsqrt(xs)


def _log(x):
    xi = lax.bitcast_convert_type(x, _I32)
    e = lax.shift_right_arithmetic(xi, 23) - 127
    mi = lax.bitwise_or(lax.bitwise_and(xi, jnp.int32(0x007FFFFF)),
                        jnp.int32(0x3F800000))
    m = lax.bitcast_convert_type(mi, _F32)
    big = m > 1.4142135
    m = jnp.where(big, m * 0.5, m)
    e = jnp.where(big, e + 1, e)
    t = (m - 1.0) / (m + 1.0)
    t2 = t * t
    p = 2.0 * t * (1.0 + t2 * (1.0 / 3.0 + t2 * (0.2 + t2 * (1.0 / 7.0))))
    return p + e.astype(_F32) * _LN2


def _tree_max(vs):
    while len(vs) > 1:
        nxt = [jnp.maximum(vs[i], vs[i + 1]) for i in range(0, len(vs) - 1, 2)]
        if len(vs) % 2:
            nxt.append(vs[-1])
        vs = nxt
    return vs[0]


def _tree_sum(vs):
    while len(vs) > 1:
        nxt = [vs[i] + vs[i + 1] for i in range(0, len(vs) - 1, 2)]
        if len(vs) % 2:
            nxt.append(vs[-1])
        vs = nxt
    return vs[0]


def _zeros8():
    return tuple(jnp.zeros((16,), _F32) for _ in range(8))


def _accum(accs, cnts, b, val):
    na, nc = [], []
    for k in range(8):
        msk = b == k
        na.append(accs[k] + jnp.where(msk, val, 0.0))
        nc.append(cnts[k] + jnp.where(msk, 1.0, 0.0))
    return tuple(na), tuple(nc)


def _halves(iota, lo_list, hi_list):
    vec = jnp.zeros((16,), _F32)
    for k in range(8):
        vec = jnp.where(iota == k, jnp.sum(lo_list[k]), vec)
        vec = jnp.where(iota == (k + 8), jnp.sum(hi_list[k]), vec)
    return vec


def _sc_body(aat_h, mcp_h, mcl_h, scp_h, scl_h, msk_h, cap_h, cal_h,
             lab_h, ia_h, im_h, isc_h, ic_h, out_h,
             aat_v, mcp_v, mcl_v, scp_v, scl_v, msk_v, cap_v, cal_v,
             lab_v, ia_v, im_v, isc_v, ic_v, part_v, sem):
    wid = lax.axis_index("s") * 2 + lax.axis_index("c")
    iota = lax.iota(_I32, 16)
    tok0 = wid * CHS

    d_a = [pltpu.async_copy(aat_h.at[:, pl.ds(tok0, CHS)], aat_v, sem),
           pltpu.async_copy(lab_h.at[pl.ds(tok0, CHS)], lab_v, sem),
           pltpu.async_copy(ia_h.at[pl.ds(tok0, CHS)], ia_v, sem)]
    d_m = [pltpu.async_copy(mcp_h.at[:, :, pl.ds(tok0, CHS)], mcp_v, sem),
           pltpu.async_copy(mcl_h.at[:, :, pl.ds(tok0, CHS)], mcl_v, sem),
           pltpu.async_copy(im_h.at[pl.ds(tok0, CHS)], im_v, sem)]
    d_s = [pltpu.async_copy(scp_h.at[:, :, pl.ds(tok0, CHS)], scp_v, sem),
           pltpu.async_copy(scl_h.at[:, :, pl.ds(tok0, CHS)], scl_v, sem),
           pltpu.async_copy(msk_h.at[:, pl.ds(tok0, CHS)], msk_v, sem),
           pltpu.async_copy(isc_h.at[pl.ds(tok0, CHS)], isc_v, sem)]
    d_c = [pltpu.async_copy(cap_h.at[:, pl.ds(tok0, CHS)], cap_v, sem),
           pltpu.async_copy(cal_h.at[:, pl.ds(tok0, CHS)], cal_v, sem),
           pltpu.async_copy(ic_h.at[pl.ds(tok0, CHS)], ic_v, sem)]
    for d in d_a:
        d.wait()

    def aat_body(i, carry):
        accs, cnts = carry
        base = i * 16
        b = ia_v[pl.ds(base, 16)]
        lbl = lab_v[pl.ds(base, 16)]
        vals = [aat_v[c, pl.ds(base, 16)] for c in range(C)]
        mx = _tree_max(vals)
        ssum = _tree_sum([jnp.exp(v - mx) for v in vals])
        g = plsc.load_gather(aat_v, [lbl, iota + base])
        ce = mx + _log(ssum) - g
        pt = jnp.exp(-ce)
        loss = 0.25 * (1.0 - pt) * (1.0 - pt) * ce
        return _accum(accs, cnts, b, loss)

    aat_accs, aat_cnts = lax.fori_loop(0, NV, aat_body, (_zeros8(), _zeros8()))
    for d in d_m:
        d.wait()

    def mc_body(i, carry):
        accs, cnts = carry
        base = i * 16
        b = im_v[pl.ds(base, 16)]
        dists = []
        for a in range(NMC):
            dx = (mcp_v[0, a, pl.ds(base, 16)]
                  - mcl_v[0, a, pl.ds(base, 16)])
            dy = (mcp_v[1, a, pl.ds(base, 16)]
                  - mcl_v[1, a, pl.ds(base, 16)])
            dz = (mcp_v[2, a, pl.ds(base, 16)]
                  - mcl_v[2, a, pl.ds(base, 16)])
            dists.append(_sqrt(dx * dx + dy * dy + dz * dz))
        return _accum(accs, cnts, b, _tree_sum(dists))

    mc_accs, mc_cnts = lax.fori_loop(0, NV, mc_body, (_zeros8(), _zeros8()))
    for d in d_s:
        d.wait()

    def sc_body(i, carry):
        accs, cnts = carry
        base = i * 16
        b = isc_v[pl.ds(base, 16)]
        dists = []
        for a in range(NSC):
            dx = (scp_v[a, 0, pl.ds(base, 16)]
                  - scl_v[a, 0, pl.ds(base, 16)])
            dy = (scp_v[a, 1, pl.ds(base, 16)]
                  - scl_v[a, 1, pl.ds(base, 16)])
            dz = (scp_v[a, 2, pl.ds(base, 16)]
                  - scl_v[a, 2, pl.ds(base, 16)])
            mv = msk_v[a, pl.ds(base, 16)]
            dists.append(_sqrt(dx * dx + dy * dy + dz * dz) * mv)
        return _accum(accs, cnts, b, _tree_sum(dists))

    sc_accs, sc_cnts = lax.fori_loop(0, NV, sc_body, (_zeros8(), _zeros8()))
    for d in d_c:
        d.wait()

    def ca_body(i, carry):
        accs, cnts = carry
        base = i * 16
        b = ic_v[pl.ds(base, 16)]
        dx = cap_v[0, pl.ds(base, 16)] - cal_v[0, pl.ds(base, 16)]
        dy = cap_v[1, pl.ds(base, 16)] - cal_v[1, pl.ds(base, 16)]
        dz = cap_v[2, pl.ds(base, 16)] - cal_v[2, pl.ds(base, 16)]
        tot = _sqrt(dx * dx + dy * dy + dz * dz)
        return _accum(accs, cnts, b, tot)

    ca_accs, ca_cnts = lax.fori_loop(0, NV, ca_body, (_zeros8(), _zeros8()))

    part_v[pl.ds(0, 16)] = _halves(iota, aat_accs, mc_accs)
    part_v[pl.ds(16, 16)] = _halves(iota, sc_accs, ca_accs)
    part_v[pl.ds(32, 16)] = _halves(iota, aat_cnts, mc_cnts)
    part_v[pl.ds(48, 16)] = _halves(iota, sc_cnts, ca_cnts)
    z = jnp.zeros((16,), _F32)
    part_v[pl.ds(64, 16)] = z
    part_v[pl.ds(80, 16)] = z
    part_v[pl.ds(96, 16)] = z
    part_v[pl.ds(112, 16)] = z
    pltpu.sync_copy(part_v, out_h.at[wid])


_mesh = plsc.VectorSubcoreMesh(core_axis_name="c", subcore_axis_name="s",
                               num_cores=2)

_sc_call = functools.partial(
    pl.kernel,
    out_type=jax.ShapeDtypeStruct((NT, 128), _F32),
    mesh=_mesh,
    compiler_params=pltpu.CompilerParams(
        needs_layout_passes=False,
        disable_bounds_checks=True,
        disable_semaphore_checks=True,
    ),
    scratch_types=[
        pltpu.VMEM((C, CHS), _F32),
        pltpu.VMEM((3, NMC, CHS), _F32),
        pltpu.VMEM((3, NMC, CHS), _F32),
        pltpu.VMEM((NSC, 3, CHS), _F32),
        pltpu.VMEM((NSC, 3, CHS), _F32),
        pltpu.VMEM((NSC, CHS), _F32),
        pltpu.VMEM((3, CHS), _F32),
        pltpu.VMEM((3, CHS), _F32),
        pltpu.VMEM((CHS,), _I32),
        pltpu.VMEM((CHS,), _I32),
        pltpu.VMEM((CHS,), _I32),
        pltpu.VMEM((CHS,), _I32),
        pltpu.VMEM((CHS,), _I32),
        pltpu.VMEM((128,), _F32),
        pltpu.SemaphoreType.DMA,
    ],
)(_sc_body)


def _tc_combine(p_ref, o_grad, o_aat, o_mc, o_sc, o_ca):
    x = p_ref[...]
    tot = jnp.sum(x, axis=0, keepdims=True)
    sums = tot[:, 0:32]
    cnts = tot[:, 32:64]
    means = sums / jnp.maximum(cnts, 1.0)
    aat = jnp.sum(means[:, 0:8]) * (1.0 / NBINS)
    mc = jnp.sum(means[:, 8:16]) * (1.0 / (NBINS * NMC))
    sc = jnp.sum(means[:, 16:24]) * (1.0 / (NBINS * NSC))
    ca = jnp.sum(means[:, 24:32]) * (1.0 / NBINS)
    grad = aat + ca + mc + 0.5 * sc
    o_grad[0, 0] = grad
    o_aat[0, 0] = aat
    o_mc[0, 0] = mc
    o_sc[0, 0] = sc
    o_ca[0, 0] = ca


_tc_call = pl.pallas_call(
    _tc_combine,
    out_shape=[jax.ShapeDtypeStruct((1, 1), _F32)] * 5,
    out_specs=[pl.BlockSpec(memory_space=pltpu.SMEM)] * 5,
)


def kernel(AAtype_pred, MCcoor_pred, SCcoor_pred, CAnoise_pred, AAtype_label,
           MCcoor_label, SCcoor_label, SCcoor_mask, CAnoise_label,
           AAtype_scatter, MCcoor_scatter, SCcoor_scatter, CAnoise_scatter):
    part = _sc_call(
        AAtype_pred.T,
        jnp.transpose(MCcoor_pred, (2, 1, 0)),
        jnp.transpose(MCcoor_label, (2, 1, 0)),
        jnp.transpose(SCcoor_pred, (1, 2, 0)),
        jnp.transpose(SCcoor_label, (1, 2, 0)),
        SCcoor_mask.T,
        CAnoise_pred.T,
        CAnoise_label.T,
        AAtype_label.astype(jnp.int32),
        AAtype_scatter.astype(jnp.int32),
        MCcoor_scatter.astype(jnp.int32),
        SCcoor_scatter.astype(jnp.int32),
        CAnoise_scatter.astype(jnp.int32),
    )
    grad, aat, mc, sc, ca = _tc_call(part)
    return (grad.reshape(()), aat.reshape(()), mc.reshape(()),
            sc.reshape(()), ca.reshape(()))


# speedup vs baseline: 1.1118x; 1.0044x over previous
import functools

import jax
import jax.numpy as jnp
from jax import lax
from jax.experimental import pallas as pl
from jax.experimental.pallas import tpu as pltpu
from jax.experimental.pallas import tpu_sc as plsc

N = 16384
C = 20
NBINS = 8
NMC = 4
NSC = 10
NT = 32
CHS = N // NT
NV = CHS // 16

_AAT = 0
_MCP = _AAT + C
_MCL = _MCP + 3 * NMC
_SCP = _MCL + 3 * NMC
_SCL = _SCP + 3 * NSC
_MSK = _SCL + 3 * NSC
_CAP = _MSK + NSC
_CAL = _CAP + 3
_NROWS = _CAL + 3

_F32 = jnp.float32
_I32 = jnp.int32
_LN2 = 0.6931471805599453


def _rsqrt(x):
    i = lax.bitcast_convert_type(x, _I32)
    i = jnp.int32(0x5F3759DF) - lax.shift_right_arithmetic(i, 1)
    y = lax.bitcast_convert_type(i, _F32)
    for _ in range(3):
        y = y * (1.5 - 0.5 * x * y * y)
    return y


def _sqrt(x):
    xs = jnp.maximum(x, 1e-30)
    return x * _rsqrt(xs)


def _log(x):
    xi = lax.bitcast_convert_type(x, _I32)
    e = lax.shift_right_arithmetic(xi, 23) - 127
    mi = lax.bitwise_or(lax.bitwise_and(xi, jnp.int32(0x007FFFFF)),
                        jnp.int32(0x3F800000))
    m = lax.bitcast_convert_type(mi, _F32)
    big = m > 1.4142135
    m = jnp.where(big, m * 0.5, m)
    e = jnp.where(big, e + 1, e)
    t = (m - 1.0) / (m + 1.0)
    t2 = t * t
    p = 2.0 * t * (1.0 + t2 * (1.0 / 3.0 + t2 * (0.2 + t2 * (1.0 / 7.0))))
    return p + e.astype(_F32) * _LN2


def _tree_max(vs):
    while len(vs) > 1:
        nxt = [jnp.maximum(vs[i], vs[i + 1]) for i in range(0, len(vs) - 1, 2)]
        if len(vs) % 2:
            nxt.append(vs[-1])
        vs = nxt
    return vs[0]


def _tree_sum(vs):
    while len(vs) > 1:
        nxt = [vs[i] + vs[i + 1] for i in range(0, len(vs) - 1, 2)]
        if len(vs) % 2:
            nxt.append(vs[-1])
        vs = nxt
    return vs[0]


def _zeros8():
    return tuple(jnp.zeros((16,), _F32) for _ in range(8))


def _accum(accs, cnts, b, val):
    na, nc = [], []
    for k in range(8):
        msk = b == k
        na.append(accs[k] + jnp.where(msk, val, 0.0))
        nc.append(cnts[k] + jnp.where(msk, 1.0, 0.0))
    return tuple(na), tuple(nc)


def _halves(iota, lo_list, hi_list):
    vec = jnp.zeros((16,), _F32)
    for k in range(8):
        vec = jnp.where(iota == k, jnp.sum(lo_list[k]), vec)
        vec = jnp.where(iota == (k + 8), jnp.sum(hi_list[k]), vec)
    return vec


def _sc_body(aat_h, mcp_h, mcl_h, scp_h, scl_h, msk_h, cap_h, cal_h,
             lab_h, ia_h, im_h, isc_h, ic_h, out_h,
             aat_v, mcp_v, mcl_v, scp_v, scl_v, msk_v, cap_v, cal_v,
             lab_v, ia_v, im_v, isc_v, ic_v, part_v, sem):
    wid = lax.axis_index("s") * 2 + lax.axis_index("c")
    iota = lax.iota(_I32, 16)
    tok0 = wid * CHS

    d_a = [pltpu.async_copy(aat_h.at[:, pl.ds(tok0, CHS)], aat_v, sem),
           pltpu.async_copy(lab_h.at[pl.ds(tok0, CHS)], lab_v, sem),
           pltpu.async_copy(ia_h.at[pl.ds(tok0, CHS)], ia_v, sem)]
    d_m = [pltpu.async_copy(mcp_h.at[:, :, pl.ds(tok0, CHS)], mcp_v, sem),
           pltpu.async_copy(mcl_h.at[:, :, pl.ds(tok0, CHS)], mcl_v, sem),
           pltpu.async_copy(im_h.at[pl.ds(tok0, CHS)], im_v, sem)]
    d_s = [pltpu.async_copy(scp_h.at[:, :, pl.ds(tok0, CHS)], scp_v, sem),
           pltpu.async_copy(scl_h.at[:, :, pl.ds(tok0, CHS)], scl_v, sem),
           pltpu.async_copy(msk_h.at[:, pl.ds(tok0, CHS)], msk_v, sem),
           pltpu.async_copy(isc_h.at[pl.ds(tok0, CHS)], isc_v, sem)]
    d_c = [pltpu.async_copy(cap_h.at[:, pl.ds(tok0, CHS)], cap_v, sem),
           pltpu.async_copy(cal_h.at[:, pl.ds(tok0, CHS)], cal_v, sem),
           pltpu.async_copy(ic_h.at[pl.ds(tok0, CHS)], ic_v, sem)]
    for d in d_a:
        d.wait()

    def aat_body(i, carry):
        accs, cnts = carry
        base = i * 16
        b = ia_v[pl.ds(base, 16)]
        lbl = lab_v[pl.ds(base, 16)]
        vals = [aat_v[c, pl.ds(base, 16)] for c in range(C)]
        mx = _tree_max(vals)
        ssum = _tree_sum([jnp.exp(v - mx) for v in vals])
        g = plsc.load_gather(aat_v, [lbl, iota + base])
        ce = mx + _log(ssum) - g
        pt = jnp.exp(-ce)
        loss = 0.25 * (1.0 - pt) * (1.0 - pt) * ce
        return _accum(accs, cnts, b, loss)

    aat_accs, aat_cnts = lax.fori_loop(0, NV, aat_body, (_zeros8(), _zeros8()))
    for d in d_m:
        d.wait()

    def mc_body(i, carry):
        accs, cnts = carry
        base = i * 16
        b = im_v[pl.ds(base, 16)]
        dists = []
        for a in range(NMC):
            dx = (mcp_v[0, a, pl.ds(base, 16)]
                  - mcl_v[0, a, pl.ds(base, 16)])
            dy = (mcp_v[1, a, pl.ds(base, 16)]
                  - mcl_v[1, a, pl.ds(base, 16)])
            dz = (mcp_v[2, a, pl.ds(base, 16)]
                  - mcl_v[2, a, pl.ds(base, 16)])
            dists.append(_sqrt(dx * dx + dy * dy + dz * dz))
        return _accum(accs, cnts, b, _tree_sum(dists))

    mc_accs, mc_cnts = lax.fori_loop(0, NV, mc_body, (_zeros8(), _zeros8()))
    for d in d_s:
        d.wait()

    def sc_body(i, carry):
        accs, cnts = carry
        base = i * 16
        b = isc_v[pl.ds(base, 16)]
        dists = []
        for a in range(NSC):
            dx = (scp_v[a, 0, pl.ds(base, 16)]
                  - scl_v[a, 0, pl.ds(base, 16)])
            dy = (scp_v[a, 1, pl.ds(base, 16)]
                  - scl_v[a, 1, pl.ds(base, 16)])
            dz = (scp_v[a, 2, pl.ds(base, 16)]
                  - scl_v[a, 2, pl.ds(base, 16)])
            mv = msk_v[a, pl.ds(base, 16)]
            dists.append(_sqrt(dx * dx + dy * dy + dz * dz) * mv)
        return _accum(accs, cnts, b, _tree_sum(dists))

    sc_accs, sc_cnts = lax.fori_loop(0, NV, sc_body, (_zeros8(), _zeros8()))
    for d in d_c:
        d.wait()

    def ca_body(i, carry):
        accs, cnts = carry
        base = i * 16
        b = ic_v[pl.ds(base, 16)]
        dx = cap_v[0, pl.ds(base, 16)] - cal_v[0, pl.ds(base, 16)]
        dy = cap_v[1, pl.ds(base, 16)] - cal_v[1, pl.ds(base, 16)]
        dz = cap_v[2, pl.ds(base, 16)] - cal_v[2, pl.ds(base, 16)]
        tot = _sqrt(dx * dx + dy * dy + dz * dz)
        return _accum(accs, cnts, b, tot)

    ca_accs, ca_cnts = lax.fori_loop(0, NV, ca_body, (_zeros8(), _zeros8()))

    part_v[pl.ds(0, 16)] = _halves(iota, aat_accs, mc_accs)
    part_v[pl.ds(16, 16)] = _halves(iota, sc_accs, ca_accs)
    part_v[pl.ds(32, 16)] = _halves(iota, aat_cnts, mc_cnts)
    part_v[pl.ds(48, 16)] = _halves(iota, sc_cnts, ca_cnts)
    z = jnp.zeros((16,), _F32)
    part_v[pl.ds(64, 16)] = z
    part_v[pl.ds(80, 16)] = z
    part_v[pl.ds(96, 16)] = z
    part_v[pl.ds(112, 16)] = z
    pltpu.sync_copy(part_v, out_h.at[wid])


_mesh = plsc.VectorSubcoreMesh(core_axis_name="c", subcore_axis_name="s",
                               num_cores=2)

_sc_call = functools.partial(
    pl.kernel,
    out_type=jax.ShapeDtypeStruct((NT, 128), _F32),
    mesh=_mesh,
    compiler_params=pltpu.CompilerParams(needs_layout_passes=False),
    scratch_types=[
        pltpu.VMEM((C, CHS), _F32),
        pltpu.VMEM((3, NMC, CHS), _F32),
        pltpu.VMEM((3, NMC, CHS), _F32),
        pltpu.VMEM((NSC, 3, CHS), _F32),
        pltpu.VMEM((NSC, 3, CHS), _F32),
        pltpu.VMEM((NSC, CHS), _F32),
        pltpu.VMEM((3, CHS), _F32),
        pltpu.VMEM((3, CHS), _F32),
        pltpu.VMEM((CHS,), _I32),
        pltpu.VMEM((CHS,), _I32),
        pltpu.VMEM((CHS,), _I32),
        pltpu.VMEM((CHS,), _I32),
        pltpu.VMEM((CHS,), _I32),
        pltpu.VMEM((128,), _F32),
        pltpu.SemaphoreType.DMA,
    ],
)(_sc_body)


def _tc_combine(p_ref, o_grad, o_aat, o_mc, o_sc, o_ca):
    x = p_ref[...]
    tot = jnp.sum(x, axis=0, keepdims=True)
    sums = tot[:, 0:32]
    cnts = tot[:, 32:64]
    means = sums / jnp.maximum(cnts, 1.0)
    aat = jnp.sum(means[:, 0:8]) * (1.0 / NBINS)
    mc = jnp.sum(means[:, 8:16]) * (1.0 / (NBINS * NMC))
    sc = jnp.sum(means[:, 16:24]) * (1.0 / (NBINS * NSC))
    ca = jnp.sum(means[:, 24:32]) * (1.0 / NBINS)
    grad = aat + ca + mc + 0.5 * sc
    o_grad[0, 0] = grad
    o_aat[0, 0] = aat
    o_mc[0, 0] = mc
    o_sc[0, 0] = sc
    o_ca[0, 0] = ca


_tc_call = pl.pallas_call(
    _tc_combine,
    out_shape=[jax.ShapeDtypeStruct((1, 1), _F32)] * 5,
    out_specs=[pl.BlockSpec(memory_space=pltpu.SMEM)] * 5,
)


def kernel(AAtype_pred, MCcoor_pred, SCcoor_pred, CAnoise_pred, AAtype_label,
           MCcoor_label, SCcoor_label, SCcoor_mask, CAnoise_label,
           AAtype_scatter, MCcoor_scatter, SCcoor_scatter, CAnoise_scatter):
    part = _sc_call(
        AAtype_pred.T,
        jnp.transpose(MCcoor_pred, (2, 1, 0)),
        jnp.transpose(MCcoor_label, (2, 1, 0)),
        jnp.transpose(SCcoor_pred, (1, 2, 0)),
        jnp.transpose(SCcoor_label, (1, 2, 0)),
        SCcoor_mask.T,
        CAnoise_pred.T,
        CAnoise_label.T,
        AAtype_label.astype(jnp.int32),
        AAtype_scatter.astype(jnp.int32),
        MCcoor_scatter.astype(jnp.int32),
        SCcoor_scatter.astype(jnp.int32),
        CAnoise_scatter.astype(jnp.int32),
    )
    grad, aat, mc, sc, ca = _tc_call(part)
    return (grad.reshape(()), aat.reshape(()), mc.reshape(()),
            sc.reshape(()), ca.reshape(()))
